# baseline (speedup 1.0000x reference)
import jax
import jax.numpy as jnp
from jax import lax
from jax.experimental import pallas as pl
from jax.experimental.pallas import tpu as pltpu
from jax.experimental.pallas import tpu_sc as plsc

_B = 4096
_L = 200
_D = 128
_ROWS = _B * _L
_NC = 2
_NS = 16
_NW = _NC * _NS
_SPW = _B // _NW
_RPW = _SPW * _L
_NPAIR = _SPW // 2
_IW = 100
_NKS = _D // 16


def _sc_body(seq_hbm, item_hbm, pos_hbm, out_hbm,
             idx, rows, pos_v, isems, gsems, ssems):
    c = lax.axis_index("c")
    s = lax.axis_index("s")
    wid = s * _NC + c
    base = wid * _RPW
    ibase = wid * (_RPW // _IW)

    pltpu.sync_copy(pos_hbm, pos_v)

    def start_idx(si, q):
        sc_ = jnp.minimum(si, _SPW - 1)
        pltpu.async_copy(seq_hbm.at[pl.ds(ibase + sc_ * 2, 2)],
                         idx.at[q], isems.at[q])

    def wait_idx(q):
        pltpu.make_async_copy(seq_hbm.at[pl.ds(ibase, 2)], idx.at[q],
                              isems.at[q]).wait()

    def start_gathers(q):
        for r in range(2):
            pltpu.async_copy(item_hbm.at[idx.at[q, r]],
                             rows.at[q, pl.ds(r * _IW, _IW)], gsems.at[q])

    def wait_gathers(q):
        for r in range(2):
            pltpu.make_async_copy(item_hbm.at[idx.at[q, r]],
                                  rows.at[q, pl.ds(r * _IW, _IW)],
                                  gsems.at[q]).wait()

    def start_scatter(si, q):
        pltpu.async_copy(rows.at[q], out_hbm.at[pl.ds(base + si * _L, _L)],
                         ssems.at[q])

    def wait_scatter(q):
        pltpu.make_async_copy(rows.at[q], out_hbm.at[pl.ds(base, _L)],
                              ssems.at[q]).wait()

    def add_pos_pair(q0, q1):
        def body_l(l, carry):
            pbase = l * _D
            pv = [pos_v[pl.ds(pbase + k * 16, 16)] for k in range(_NKS)]
            for q in (q0, q1):
                for k in range(_NKS):
                    plsc.addupdate(rows.at[q, l, pl.ds(k * 16, 16)], pv[k])
            return carry
        lax.fori_loop(0, _L, body_l, 0, unroll=2)

    for q in range(4):
        start_idx(q, q)
    wait_idx(0)
    wait_idx(1)
    start_gathers(0)
    start_gathers(1)

    def pair_body(p, carry):
        for grp in range(2):
            pp = 2 * p + grp
            b0, b1 = (0, 1) if grp == 0 else (2, 3)
            s0 = 2 * pp
            n0, n1 = (2, 3) if grp == 0 else (0, 1)

            @pl.when(pp >= 1)
            def _():
                wait_scatter(n0)
                wait_scatter(n1)

            @pl.when(pp < _NPAIR - 1)
            def _():
                wait_idx(n0)
                wait_idx(n1)
                start_gathers(n0)
                start_gathers(n1)

            wait_gathers(b0)
            wait_gathers(b1)
            start_idx(s0 + 4, b0)
            start_idx(s0 + 5, b1)
            add_pos_pair(b0, b1)
            start_scatter(s0, b0)
            start_scatter(s0 + 1, b1)
        return carry

    lax.fori_loop(0, _NPAIR // 2, pair_body, 0)

    wait_scatter(2)
    wait_scatter(3)
    for q in range(4):
        wait_idx(q)


@jax.jit
def _sc_embed(seq_view, item_table, pos_flat):
    mesh = plsc.VectorSubcoreMesh(
        core_axis_name="c", subcore_axis_name="s",
        num_cores=_NC, num_subcores=_NS)
    return pl.kernel(
        _sc_body,
        out_type=jax.ShapeDtypeStruct((_ROWS, _D), jnp.float32),
        mesh=mesh,
        scratch_types=[
            pltpu.VMEM((4, 2, _IW), jnp.int32),
            pltpu.VMEM((4, _L, _D), jnp.float32),
            pltpu.VMEM((_L * _D,), jnp.float32),
            pltpu.SemaphoreType.DMA((4,)),
            pltpu.SemaphoreType.DMA((4,)),
            pltpu.SemaphoreType.DMA((4,)),
        ],
    )(seq_view, item_table, pos_flat)


def kernel(sequence, item_table, pos_table):
    seq_view = sequence.reshape(_ROWS // _IW, _IW).astype(jnp.int32)
    out = _sc_embed(seq_view, item_table, pos_table.reshape(-1))
    return out.reshape(_B, _L, _D)


# speedup vs baseline: 1.0007x; 1.0007x over previous
import jax
import jax.numpy as jnp
from jax import lax
from jax.experimental import pallas as pl
from jax.experimental.pallas import tpu as pltpu
from jax.experimental.pallas import tpu_sc as plsc

_B = 4096
_L = 200
_D = 128
_ROWS = _B * _L
_NC = 2
_NS = 16
_NW = _NC * _NS
_SPW = _B // _NW
_RPW = _SPW * _L
_NPAIR = _SPW // 2
_TPAIR = _B // 2
_IW = 100
_NKS = _D // 16


def _sc_body(seq_hbm, item_hbm, pos_hbm, out_hbm,
             idx, rows, pos_v, isems, gsems, ssems):
    c = lax.axis_index("c")
    s = lax.axis_index("s")
    wid = s * _NC + c
    pbase0 = wid * _NPAIR
    ibase = pbase0 * 4

    pltpu.sync_copy(pos_hbm, pos_v)

    def start_idx(pp, g):
        cc = jnp.minimum(pp, _NPAIR - 1)
        pltpu.async_copy(seq_hbm.at[pl.ds(ibase + cc * 4, 4)],
                         idx.at[g], isems.at[g])

    def wait_idx(g):
        pltpu.make_async_copy(seq_hbm.at[pl.ds(ibase, 4)], idx.at[g],
                              isems.at[g]).wait()

    def start_gathers(g):
        for r in range(4):
            pltpu.async_copy(item_hbm.at[idx.at[g, r]],
                             rows.at[g, r // 2, pl.ds((r % 2) * _IW, _IW)],
                             gsems.at[g])

    def wait_gathers(g):
        for r in range(4):
            pltpu.make_async_copy(item_hbm.at[idx.at[g, r]],
                                  rows.at[g, r // 2, pl.ds((r % 2) * _IW, _IW)],
                                  gsems.at[g]).wait()

    def start_scatter(pp, g):
        pltpu.async_copy(rows.at[g], out_hbm.at[pbase0 + pp], ssems.at[g])

    def wait_scatter(g):
        pltpu.make_async_copy(rows.at[g], out_hbm.at[pbase0], ssems.at[g]).wait()

    def add_pos_pair(g):
        def body_l(l, carry):
            pb = l * _D
            pv = [pos_v[pl.ds(pb + k * 16, 16)] for k in range(_NKS)]
            for q in range(2):
                for k in range(_NKS):
                    plsc.addupdate(rows.at[g, q, l, pl.ds(k * 16, 16)], pv[k])
            return carry
        lax.fori_loop(0, _L, body_l, 0, unroll=2)

    start_idx(0, 0)
    start_idx(1, 1)
    wait_idx(0)
    start_gathers(0)

    def pair_body(p, carry):
        for g in range(2):
            pp = 2 * p + g
            og = 1 - g

            @pl.when(pp >= 1)
            def _():
                wait_scatter(og)

            @pl.when(pp < _NPAIR - 1)
            def _():
                wait_idx(og)
                start_gathers(og)

            wait_gathers(g)
            start_idx(pp + 2, g)
            add_pos_pair(g)
            start_scatter(pp, g)
        return carry

    lax.fori_loop(0, _NPAIR // 2, pair_body, 0)

    wait_scatter(1)
    wait_idx(0)
    wait_idx(1)


@jax.jit
def _sc_embed(seq_view, item_table, pos_flat):
    mesh = plsc.VectorSubcoreMesh(
        core_axis_name="c", subcore_axis_name="s",
        num_cores=_NC, num_subcores=_NS)
    return pl.kernel(
        _sc_body,
        out_type=jax.ShapeDtypeStruct((_TPAIR, 2, _L, _D), jnp.float32),
        mesh=mesh,
        scratch_types=[
            pltpu.VMEM((2, 4, _IW), jnp.int32),
            pltpu.VMEM((2, 2, _L, _D), jnp.float32),
            pltpu.VMEM((_L * _D,), jnp.float32),
            pltpu.SemaphoreType.DMA((2,)),
            pltpu.SemaphoreType.DMA((2,)),
            pltpu.SemaphoreType.DMA((2,)),
        ],
    )(seq_view, item_table, pos_flat)


def kernel(sequence, item_table, pos_table):
    seq_view = sequence.reshape(_ROWS // _IW, _IW).astype(jnp.int32)
    out = _sc_embed(seq_view, item_table, pos_table.reshape(-1))
    return out.reshape(_B, _L, _D)
